# baseline (speedup 1.0000x reference)
import functools

import jax
import jax.numpy as jnp
from jax import lax
from jax.experimental import pallas as pl
from jax.experimental.pallas import tpu as pltpu
from jax.experimental.pallas import tpu_sc as plsc

D_EMBED = 64
SCALE = float(64 ** 0.5)

NUM_CORES = 2
NUM_SUBCORES = 16
NW = NUM_CORES * NUM_SUBCORES
CHUNK = 128


def _build_sc_gather(n_chunks: int):
    mesh = plsc.VectorSubcoreMesh(core_axis_name="c", subcore_axis_name="s")

    @functools.partial(
        pl.kernel,
        mesh=mesh,
        out_type=jax.ShapeDtypeStruct((NW, n_chunks, CHUNK, D_EMBED), jnp.float32),
        scratch_types=[
            pltpu.VMEM((CHUNK,), jnp.int32),
            pltpu.VMEM((CHUNK, D_EMBED), jnp.float32),
            pltpu.SemaphoreType.DMA,
        ],
        compiler_params=pltpu.CompilerParams(use_tc_tiling_on_sc=False),
    )
    def gather_kernel(table_hbm, idx_hbm, out_hbm, idx_v, rows_v, sem):
        wid = lax.axis_index("s") * NUM_CORES + lax.axis_index("c")

        def chunk_step(g, carry):
            pltpu.sync_copy(idx_hbm.at[wid, g], idx_v)
            pltpu.async_copy(table_hbm.at[idx_v], rows_v, sem).wait()

            def scale_row(i, c):
                for q in range(D_EMBED // 16):
                    sl = pl.ds(q * 16, 16)
                    rows_v[i, sl] = rows_v[i, sl] * SCALE
                return c

            lax.fori_loop(0, CHUNK, scale_row, 0)
            pltpu.sync_copy(rows_v, out_hbm.at[wid, g])
            return carry

        lax.fori_loop(0, n_chunks, chunk_step, 0)

    return gather_kernel


def kernel(inp_tokens, emb_table):
    b, s = inp_tokens.shape
    total = b * s
    assert total % (NW * CHUNK) == 0
    n_chunks = total // (NW * CHUNK)
    idx = inp_tokens.reshape(NW, n_chunks, CHUNK)
    out = _build_sc_gather(n_chunks)(emb_table, idx)
    return out.reshape(b, s, D_EMBED)


# speedup vs baseline: 1.2334x; 1.2334x over previous
import functools

import jax
import jax.numpy as jnp
from jax import lax
from jax.experimental import pallas as pl
from jax.experimental.pallas import tpu as pltpu
from jax.experimental.pallas import tpu_sc as plsc

D_EMBED = 64
SCALE = float(64 ** 0.5)

NUM_CORES = 2
NUM_SUBCORES = 16
NW = NUM_CORES * NUM_SUBCORES
CHUNK = 128
NBUF = 4


def _build_sc_gather(n_chunks: int):
    mesh = plsc.VectorSubcoreMesh(core_axis_name="c", subcore_axis_name="s")

    @functools.partial(
        pl.kernel,
        mesh=mesh,
        out_type=jax.ShapeDtypeStruct((NW, n_chunks, CHUNK, D_EMBED), jnp.float32),
        scratch_types=[
            pltpu.VMEM((n_chunks, CHUNK), jnp.int32),
            pltpu.VMEM((NBUF, CHUNK, D_EMBED), jnp.float32),
            [pltpu.SemaphoreType.DMA] * NBUF,
            [pltpu.SemaphoreType.DMA] * NBUF,
        ],
        compiler_params=pltpu.CompilerParams(use_tc_tiling_on_sc=False),
    )
    def gather_kernel(table_hbm, idx_hbm, out_hbm, idx_v, rows_v, gsems, osems):
        wid = lax.axis_index("s") * NUM_CORES + lax.axis_index("c")
        pltpu.sync_copy(idx_hbm.at[wid], idx_v)

        def start_gather(g, b):
            pltpu.async_copy(table_hbm.at[idx_v.at[g]], rows_v.at[b], gsems[b])

        def wait_gather(b):
            pltpu.make_async_copy(
                table_hbm.at[idx_v.at[0]], rows_v.at[b], gsems[b]
            ).wait()

        def scale(b):
            def _scale_row(i, c):
                for q in range(D_EMBED // 16):
                    sl = pl.ds(q * 16, 16)
                    rows_v[b, i, sl] = rows_v[b, i, sl] * SCALE
                return c

            lax.fori_loop(0, CHUNK, _scale_row, 0)

        def start_store(g, b):
            pltpu.async_copy(rows_v.at[b], out_hbm.at[wid, g], osems[b])

        def wait_store(b):
            pltpu.make_async_copy(rows_v.at[b], out_hbm.at[wid, 0], osems[b]).wait()

        def chunk_body(g, b, wait_out: bool, lookahead: bool):
            wait_gather(b)
            scale(b)
            start_store(g, b)
            if lookahead:
                f = (b + 2) % NBUF
                if wait_out:
                    wait_store(f)
                start_gather(g + 2, f)

        start_gather(0, 0)
        start_gather(1, 1)
        chunk_body(0, 0, wait_out=False, lookahead=True)
        chunk_body(1, 1, wait_out=False, lookahead=True)
        chunk_body(2, 2, wait_out=True, lookahead=True)
        chunk_body(3, 3, wait_out=True, lookahead=True)

        def quad(k, carry):
            for b in range(NBUF):
                chunk_body(k + b, b, wait_out=True, lookahead=True)
            return carry

        lax.fori_loop(1, n_chunks // NBUF - 1, lambda q, c: quad(q * NBUF, c), 0)

        e = n_chunks - NBUF
        chunk_body(e + 0, 0, wait_out=True, lookahead=True)
        chunk_body(e + 1, 1, wait_out=True, lookahead=True)
        chunk_body(e + 2, 2, wait_out=False, lookahead=False)
        chunk_body(e + 3, 3, wait_out=False, lookahead=False)
        for b in range(NBUF):
            wait_store(b)

    return gather_kernel


def kernel(inp_tokens, emb_table):
    b, s = inp_tokens.shape
    total = b * s
    assert total % (NW * CHUNK) == 0
    n_chunks = total // (NW * CHUNK)
    idx = inp_tokens.reshape(NW, n_chunks, CHUNK)
    out = _build_sc_gather(n_chunks)(emb_table, idx)
    return out.reshape(b, s, D_EMBED)


# speedup vs baseline: 1.2532x; 1.0161x over previous
import functools

import jax
import jax.numpy as jnp
from jax import lax
from jax.experimental import pallas as pl
from jax.experimental.pallas import tpu as pltpu
from jax.experimental.pallas import tpu_sc as plsc

D_EMBED = 64
SCALE = float(64 ** 0.5)

NUM_CORES = 2
NUM_SUBCORES = 16
NW = NUM_CORES * NUM_SUBCORES
CHUNK = 128
NBUF = 4
UNROLL = 8


def _build_sc_gather(n_chunks: int):
    mesh = plsc.VectorSubcoreMesh(core_axis_name="c", subcore_axis_name="s")

    @functools.partial(
        pl.kernel,
        mesh=mesh,
        out_type=jax.ShapeDtypeStruct((NW, n_chunks, CHUNK, D_EMBED), jnp.float32),
        scratch_types=[
            pltpu.VMEM((n_chunks, CHUNK), jnp.int32),
            pltpu.VMEM((NBUF, CHUNK, D_EMBED), jnp.float32),
            [pltpu.SemaphoreType.DMA] * NBUF,
            [pltpu.SemaphoreType.DMA] * NBUF,
        ],
        compiler_params=pltpu.CompilerParams(use_tc_tiling_on_sc=False),
    )
    def gather_kernel(table_hbm, idx_hbm, out_hbm, idx_v, rows_v, gsems, osems):
        wid = lax.axis_index("s") * NUM_CORES + lax.axis_index("c")
        pltpu.sync_copy(idx_hbm.at[wid], idx_v)

        def start_gather(g, b):
            pltpu.async_copy(table_hbm.at[idx_v.at[g]], rows_v.at[b], gsems[b])

        def wait_gather(b):
            pltpu.make_async_copy(
                table_hbm.at[idx_v.at[0]], rows_v.at[b], gsems[b]
            ).wait()

        def scale(b):
            def _scale_rows(i, c):
                base = i * UNROLL
                for r in range(UNROLL):
                    for q in range(D_EMBED // 16):
                        sl = pl.ds(q * 16, 16)
                        rows_v[b, base + r, sl] = rows_v[b, base + r, sl] * SCALE
                return c

            lax.fori_loop(0, CHUNK // UNROLL, _scale_rows, 0)

        def start_store(g, b):
            pltpu.async_copy(rows_v.at[b], out_hbm.at[wid, g], osems[b])

        def wait_store(b):
            pltpu.make_async_copy(rows_v.at[b], out_hbm.at[wid, 0], osems[b]).wait()

        def chunk_body(g, b, wait_out: bool, lookahead: bool):
            wait_gather(b)
            scale(b)
            start_store(g, b)
            if lookahead:
                f = (b + 2) % NBUF
                if wait_out:
                    wait_store(f)
                start_gather(g + 2, f)

        start_gather(0, 0)
        start_gather(1, 1)
        chunk_body(0, 0, wait_out=False, lookahead=True)
        chunk_body(1, 1, wait_out=False, lookahead=True)
        chunk_body(2, 2, wait_out=True, lookahead=True)
        chunk_body(3, 3, wait_out=True, lookahead=True)

        def quad(k, carry):
            for b in range(NBUF):
                chunk_body(k + b, b, wait_out=True, lookahead=True)
            return carry

        lax.fori_loop(1, n_chunks // NBUF - 1, lambda q, c: quad(q * NBUF, c), 0)

        e = n_chunks - NBUF
        chunk_body(e + 0, 0, wait_out=True, lookahead=True)
        chunk_body(e + 1, 1, wait_out=True, lookahead=True)
        chunk_body(e + 2, 2, wait_out=False, lookahead=False)
        chunk_body(e + 3, 3, wait_out=False, lookahead=False)
        for b in range(NBUF):
            wait_store(b)

    return gather_kernel


def kernel(inp_tokens, emb_table):
    b, s = inp_tokens.shape
    total = b * s
    assert total % (NW * CHUNK) == 0
    n_chunks = total // (NW * CHUNK)
    idx = inp_tokens.reshape(NW, n_chunks, CHUNK)
    out = _build_sc_gather(n_chunks)(emb_table, idx)
    return out.reshape(b, s, D_EMBED)


# speedup vs baseline: 1.5605x; 1.2452x over previous
import functools

import jax
import jax.numpy as jnp
from jax import lax
from jax.experimental import pallas as pl
from jax.experimental.pallas import tpu as pltpu
from jax.experimental.pallas import tpu_sc as plsc

D_EMBED = 64
D_PAD = 128
SCALE = float(64 ** 0.5)

NUM_CORES = 2
NUM_SUBCORES = 16
NW = NUM_CORES * NUM_SUBCORES
CHUNK = 128
NBUF = 4
NOBUF = 2
UNROLL = 8


def _build_sc_gather(n_chunks: int):
    mesh = plsc.VectorSubcoreMesh(core_axis_name="c", subcore_axis_name="s")
    total = NW * n_chunks * CHUNK

    @functools.partial(
        pl.kernel,
        mesh=mesh,
        out_type=jax.ShapeDtypeStruct((total, D_EMBED), jnp.float32),
        scratch_types=[
            pltpu.VMEM((n_chunks, CHUNK), jnp.int32),
            pltpu.VMEM((NBUF, CHUNK, D_PAD), jnp.float32),
            pltpu.VMEM((NOBUF, CHUNK, D_EMBED), jnp.float32),
            [pltpu.SemaphoreType.DMA] * NBUF,
            [pltpu.SemaphoreType.DMA] * NOBUF,
        ],
    )
    def gather_kernel(
        table_hbm, idx_hbm, out_hbm, idx_v, rows_v, pack_v, gsems, osems
    ):
        wid = lax.axis_index("s") * NUM_CORES + lax.axis_index("c")
        base = wid * (n_chunks * CHUNK)
        pltpu.sync_copy(idx_hbm.at[pl.ds(wid * n_chunks, n_chunks)], idx_v)

        def start_gather(g, b):
            pltpu.async_copy(table_hbm.at[idx_v.at[g]], rows_v.at[b], gsems[b])

        def wait_gather(b):
            pltpu.make_async_copy(
                table_hbm.at[idx_v.at[0]], rows_v.at[b], gsems[b]
            ).wait()

        def scale(b, o):
            def _scale_rows(i, c):
                r0 = i * UNROLL
                for r in range(UNROLL):
                    for q in range(D_EMBED // 16):
                        sl = pl.ds(q * 16, 16)
                        pack_v[o, r0 + r, sl] = rows_v[b, r0 + r, sl] * SCALE
                return c

            lax.fori_loop(0, CHUNK // UNROLL, _scale_rows, 0)

        def start_store(g, o):
            pltpu.async_copy(
                pack_v.at[o], out_hbm.at[pl.ds(base + g * CHUNK, CHUNK)], osems[o]
            )

        def wait_store(o):
            pltpu.make_async_copy(
                pack_v.at[o], out_hbm.at[pl.ds(base, CHUNK)], osems[o]
            ).wait()

        def chunk_body(g, b, o, wait_out: bool, lookahead: bool):
            wait_gather(b)
            if wait_out:
                wait_store(o)
            scale(b, o)
            start_store(g, o)
            if lookahead:
                start_gather(g + 2, (b + 2) % NBUF)

        start_gather(0, 0)
        start_gather(1, 1)
        chunk_body(0, 0, 0, wait_out=False, lookahead=True)
        chunk_body(1, 1, 1, wait_out=False, lookahead=True)
        chunk_body(2, 2, 0, wait_out=True, lookahead=True)
        chunk_body(3, 3, 1, wait_out=True, lookahead=True)

        def quad(k, carry):
            for b in range(NBUF):
                chunk_body(k + b, b, b % NOBUF, wait_out=True, lookahead=True)
            return carry

        lax.fori_loop(1, n_chunks // NBUF - 1, lambda q, c: quad(q * NBUF, c), 0)

        e = n_chunks - NBUF
        chunk_body(e + 0, 0, 0, wait_out=True, lookahead=True)
        chunk_body(e + 1, 1, 1, wait_out=True, lookahead=True)
        chunk_body(e + 2, 2, 0, wait_out=True, lookahead=False)
        chunk_body(e + 3, 3, 1, wait_out=True, lookahead=False)
        for o in range(NOBUF):
            wait_store(o)

    return gather_kernel


def kernel(inp_tokens, emb_table):
    b, s = inp_tokens.shape
    total = b * s
    assert total % (NW * CHUNK) == 0
    n_chunks = total // (NW * CHUNK)
    table_pad = jnp.pad(emb_table, ((0, 0), (0, D_PAD - D_EMBED)))
    idx = inp_tokens.reshape(total // CHUNK, CHUNK)
    out = _build_sc_gather(n_chunks)(table_pad, idx)
    return out.reshape(b, s, D_EMBED)
